# baseline (speedup 1.0000x reference)
import functools

import jax
import jax.numpy as jnp
from jax import lax
from jax.experimental import pallas as pl
from jax.experimental.pallas import tpu as pltpu
from jax.experimental.pallas import tpu_sc as plsc

N = 10000
E = 320000
IN = 128
H = 64

NC = 2
NS = 16
NW = NC * NS
CHUNK = 128
CPW = 80
E_PAD = NW * CPW * CHUNK
ROWS_PT = 640
N_PAD = NS * ROWS_PT
DUMMY = N_PAD - 1
W1AGG = 80
W2AGG = 16


def _tc1_body(x_ref, wl_ref, wr_ref, b1_ref, p1_ref, r1_ref):
    x = x_ref[...]
    p1 = jnp.dot(x, wl_ref[...], preferred_element_type=jnp.float32)
    ones = jnp.ones((N, 1), jnp.float32)
    zpad = jnp.zeros((N, W1AGG - H - 1), jnp.float32)
    p1_ref[...] = jnp.concatenate([p1, ones, zpad], axis=1)
    r1_ref[...] = (
        jnp.dot(x, wr_ref[...], preferred_element_type=jnp.float32) + b1_ref[...]
    )


def _tc1(x, W1_l, W1_r, b1):
    return pl.pallas_call(
        _tc1_body,
        out_shape=[
            jax.ShapeDtypeStruct((N, W1AGG), jnp.float32),
            jax.ShapeDtypeStruct((N, H), jnp.float32),
        ],
    )(x, W1_l, W1_r, b1)


def _tc2_body(s1p_ref, r1_ref, w2l_ref, w2r_ref, b2_ref, p2_ref, aux_ref):
    s1 = s1p_ref[0, :N, :] + s1p_ref[1, :N, :]
    deg = jnp.maximum(s1[:, H:H + 1], 1.0)
    rdeg = 1.0 / deg
    h = jnp.maximum(s1[:, :H] * rdeg + r1_ref[...], 0.0)
    p2 = jnp.dot(h, w2l_ref[...], preferred_element_type=jnp.float32)
    r2b = jnp.dot(h, w2r_ref[...], preferred_element_type=jnp.float32) + b2_ref[...]
    p2_ref[...] = jnp.concatenate(
        [p2, jnp.zeros((N, W2AGG - 1), jnp.float32)], axis=1)
    aux_ref[...] = jnp.concatenate(
        [rdeg, r2b, jnp.zeros((N, 6), jnp.float32)], axis=1)


def _tc2(s1p, r1b, W2_l, W2_r, b2):
    return pl.pallas_call(
        _tc2_body,
        out_shape=[
            jax.ShapeDtypeStruct((N, W2AGG), jnp.float32),
            jax.ShapeDtypeStruct((N, 8), jnp.float32),
        ],
    )(s1p, r1b, W2_l, W2_r, b2)


def _tc3_body(s2p_ref, aux_ref, out_ref):
    s2 = s2p_ref[0, :N, 0:1] + s2p_ref[1, :N, 0:1]
    out_ref[...] = s2 * aux_ref[:, 0:1] + aux_ref[:, 1:2]


def _tc3(s2p, aux):
    return pl.pallas_call(
        _tc3_body,
        out_shape=jax.ShapeDtypeStruct((N, 1), jnp.float32),
    )(s2p, aux)


def _make_sc_agg(width):
    mesh = plsc.VectorSubcoreMesh(
        core_axis_name="c", subcore_axis_name="s", num_cores=NC, num_subcores=NS)
    nzero = ROWS_PT // CHUNK

    @functools.partial(
        pl.kernel,
        mesh=mesh,
        compiler_params=pltpu.CompilerParams(use_tc_tiling_on_sc=False),
        out_type=jax.ShapeDtypeStruct((NC, N_PAD, width), jnp.float32),
        scratch_types=[
            pltpu.VMEM((CPW, CHUNK), jnp.int32),
            pltpu.VMEM((CPW, CHUNK), jnp.int32),
            pltpu.VMEM((CHUNK, width), jnp.float32),
            pltpu.VMEM((CHUNK, width), jnp.float32),
            pltpu.VMEM_SHARED((N_PAD, width), jnp.float32),
            pltpu.SemaphoreType.DMA,
            pltpu.SemaphoreType.DMA,
        ],
    )
    def sc_agg(table_hbm, src_hbm, dst_hbm, out_hbm, src_v, dst_v, rows0,
               rows1, acc, gsem0, gsem1):
        c = lax.axis_index("c")
        s = lax.axis_index("s")
        wid = s * NC + c
        base = s * ROWS_PT

        def _zrow(i, _):
            def _zcol(j, _):
                rows0[i, pl.ds(j * 16, 16)] = jnp.zeros((16,), jnp.float32)
                return 0
            return lax.fori_loop(0, width // 16, _zcol, 0)
        lax.fori_loop(0, CHUNK, _zrow, 0)
        for k in range(nzero):
            pltpu.sync_copy(rows0, acc.at[pl.ds(base + k * CHUNK, CHUNK), :])
        plsc.subcore_barrier()

        pltpu.sync_copy(src_hbm.at[wid], src_v)
        pltpu.sync_copy(dst_hbm.at[wid], dst_v)

        def _wait(rows, gsem):
            pltpu.make_async_copy(table_hbm.at[src_v.at[0]], rows, gsem).wait()

        pltpu.async_copy(table_hbm.at[src_v.at[0]], rows0, gsem0)

        def _edge_pair(jj, _):
            j0 = 2 * jj
            j1 = j0 + 1
            _wait(rows0, gsem0)
            pltpu.async_copy(table_hbm.at[src_v.at[j1]], rows1, gsem1)
            pltpu.sync_copy(rows0, acc.at[dst_v.at[j0]], add=True)
            jn = jnp.minimum(j0 + 2, CPW - 1)
            _wait(rows1, gsem1)
            pltpu.async_copy(table_hbm.at[src_v.at[jn]], rows0, gsem0)
            pltpu.sync_copy(rows1, acc.at[dst_v.at[j1]], add=True)
            return 0
        lax.fori_loop(0, CPW // 2, _edge_pair, 0)
        _wait(rows0, gsem0)
        plsc.subcore_barrier()

        pltpu.sync_copy(acc.at[pl.ds(base, ROWS_PT), :],
                        out_hbm.at[c, pl.ds(base, ROWS_PT), :])

    return sc_agg


_sc_agg_l1 = _make_sc_agg(W1AGG)
_sc_agg_l2 = _make_sc_agg(W2AGG)


def kernel(x, edge_index, W1_l, W1_r, b1, W2_l, W2_r, b2):
    pad = E_PAD - E
    src = jnp.concatenate([edge_index[0], jnp.zeros((pad,), jnp.int32)])
    dst = jnp.concatenate([edge_index[1], jnp.full((pad,), DUMMY, jnp.int32)])
    src = src.reshape(NW, CPW, CHUNK)
    dst = dst.reshape(NW, CPW, CHUNK)

    p1ext, r1b = _tc1(x, W1_l, W1_r, b1.reshape(1, H))
    s1p = _sc_agg_l1(p1ext, src, dst)
    p2ext, aux = _tc2(s1p, r1b, W2_l, W2_r, b2.reshape(1, 1))
    s2p = _sc_agg_l2(p2ext, src, dst)
    return _tc3(s2p, aux)


# speedup vs baseline: 1.0532x; 1.0532x over previous
import functools

import jax
import jax.numpy as jnp
from jax import lax
from jax.experimental import pallas as pl
from jax.experimental.pallas import tpu as pltpu
from jax.experimental.pallas import tpu_sc as plsc

N = 10000
E = 320000
IN = 128
H = 64

NC = 2
NS = 16
NW = NC * NS
CHUNK = 128
CPW = 80
E_PAD = NW * CPW * CHUNK
ROWS_PT = 640
N_PAD = NS * ROWS_PT
DUMMY = N_PAD - 1
W1AGG = 80
W2AGG = 16
NBURST = 4


def _tc1_body(x_ref, wl_ref, wr_ref, b1_ref, p1_ref, r1_ref):
    x = x_ref[...]
    p1 = jnp.dot(x, wl_ref[...], preferred_element_type=jnp.float32)
    ones = jnp.ones((N, 1), jnp.float32)
    zpad = jnp.zeros((N, W1AGG - H - 1), jnp.float32)
    p1_ref[...] = jnp.concatenate([p1, ones, zpad], axis=1)
    r1_ref[...] = (
        jnp.dot(x, wr_ref[...], preferred_element_type=jnp.float32) + b1_ref[...]
    )


def _tc1(x, W1_l, W1_r, b1):
    return pl.pallas_call(
        _tc1_body,
        out_shape=[
            jax.ShapeDtypeStruct((N, W1AGG), jnp.float32),
            jax.ShapeDtypeStruct((N, H), jnp.float32),
        ],
    )(x, W1_l, W1_r, b1)


def _tc2_body(s1p_ref, r1_ref, w2l_ref, w2r_ref, b2_ref, p2_ref, aux_ref):
    s1 = s1p_ref[0, :N, :] + s1p_ref[1, :N, :]
    deg = jnp.maximum(s1[:, H:H + 1], 1.0)
    rdeg = 1.0 / deg
    h = jnp.maximum(s1[:, :H] * rdeg + r1_ref[...], 0.0)
    p2 = jnp.dot(h, w2l_ref[...], preferred_element_type=jnp.float32)
    r2b = jnp.dot(h, w2r_ref[...], preferred_element_type=jnp.float32) + b2_ref[...]
    p2_ref[...] = jnp.concatenate(
        [p2, jnp.zeros((N, W2AGG - 1), jnp.float32)], axis=1)
    aux_ref[...] = jnp.concatenate(
        [rdeg, r2b, jnp.zeros((N, 6), jnp.float32)], axis=1)


def _tc2(s1p, r1b, W2_l, W2_r, b2):
    return pl.pallas_call(
        _tc2_body,
        out_shape=[
            jax.ShapeDtypeStruct((N, W2AGG), jnp.float32),
            jax.ShapeDtypeStruct((N, 8), jnp.float32),
        ],
    )(s1p, r1b, W2_l, W2_r, b2)


def _tc3_body(s2p_ref, aux_ref, out_ref):
    s2 = s2p_ref[0, :N, 0:1] + s2p_ref[1, :N, 0:1]
    out_ref[...] = s2 * aux_ref[:, 0:1] + aux_ref[:, 1:2]


def _tc3(s2p, aux):
    return pl.pallas_call(
        _tc3_body,
        out_shape=jax.ShapeDtypeStruct((N, 1), jnp.float32),
    )(s2p, aux)


def _make_sc_agg(width):
    mesh = plsc.VectorSubcoreMesh(
        core_axis_name="c", subcore_axis_name="s", num_cores=NC, num_subcores=NS)
    nzero = ROWS_PT // CHUNK

    @functools.partial(
        pl.kernel,
        mesh=mesh,
        compiler_params=pltpu.CompilerParams(use_tc_tiling_on_sc=False),
        out_type=jax.ShapeDtypeStruct((NC, N_PAD, width), jnp.float32),
        scratch_types=[
            pltpu.VMEM((CPW, CHUNK), jnp.int32),
            pltpu.VMEM((CPW, CHUNK), jnp.int32),
            pltpu.VMEM((NBURST, CHUNK, width), jnp.float32),
            pltpu.VMEM_SHARED((N_PAD, width), jnp.float32),
            pltpu.SemaphoreType.DMA,
            pltpu.SemaphoreType.DMA,
        ],
    )
    def sc_agg(table_hbm, src_hbm, dst_hbm, out_hbm, src_v, dst_v, rows,
               acc, gsem, ssem):
        c = lax.axis_index("c")
        s = lax.axis_index("s")
        wid = s * NC + c
        base = s * ROWS_PT

        zbuf = rows.at[0]
        def _zrow(i, _):
            def _zcol(j, _):
                zbuf[i, pl.ds(j * 16, 16)] = jnp.zeros((16,), jnp.float32)
                return 0
            return lax.fori_loop(0, width // 16, _zcol, 0)
        lax.fori_loop(0, CHUNK, _zrow, 0)
        for k in range(nzero):
            pltpu.sync_copy(zbuf, acc.at[pl.ds(base + k * CHUNK, CHUNK), :])
        plsc.subcore_barrier()

        pltpu.sync_copy(src_hbm.at[wid], src_v)
        pltpu.sync_copy(dst_hbm.at[wid], dst_v)

        def _edge_group(g, _):
            jb = g * NBURST
            for b in range(NBURST):
                pltpu.async_copy(
                    table_hbm.at[src_v.at[jb + b]], rows.at[b], gsem)
            for b in range(NBURST):
                pltpu.make_async_copy(
                    table_hbm.at[src_v.at[jb]], rows.at[b], gsem).wait()
            for b in range(NBURST):
                pltpu.async_copy(
                    rows.at[b], acc.at[dst_v.at[jb + b]], ssem, add=True)
            for b in range(NBURST):
                pltpu.make_async_copy(
                    rows.at[b], acc.at[dst_v.at[jb]], ssem).wait()
            return 0
        lax.fori_loop(0, CPW // NBURST, _edge_group, 0)
        plsc.subcore_barrier()

        pltpu.sync_copy(acc.at[pl.ds(base, ROWS_PT), :],
                        out_hbm.at[c, pl.ds(base, ROWS_PT), :])

    return sc_agg


_sc_agg_l1 = _make_sc_agg(W1AGG)
_sc_agg_l2 = _make_sc_agg(W2AGG)


def kernel(x, edge_index, W1_l, W1_r, b1, W2_l, W2_r, b2):
    pad = E_PAD - E
    src = jnp.concatenate([edge_index[0], jnp.zeros((pad,), jnp.int32)])
    dst = jnp.concatenate([edge_index[1], jnp.full((pad,), DUMMY, jnp.int32)])
    src = src.reshape(NW, CPW, CHUNK)
    dst = dst.reshape(NW, CPW, CHUNK)

    p1ext, r1b = _tc1(x, W1_l, W1_r, b1.reshape(1, H))
    s1p = _sc_agg_l1(p1ext, src, dst)
    p2ext, aux = _tc2(s1p, r1b, W2_l, W2_r, b2.reshape(1, 1))
    s2p = _sc_agg_l2(p2ext, src, dst)
    return _tc3(s2p, aux)


# speedup vs baseline: 1.0532x; 1.0000x over previous
import functools

import jax
import jax.numpy as jnp
from jax import lax
from jax.experimental import pallas as pl
from jax.experimental.pallas import tpu as pltpu
from jax.experimental.pallas import tpu_sc as plsc

N = 10000
E = 320000
IN = 128
H = 64

NC = 2
NS = 16
NW = NC * NS
CHUNK = 128
CPW = 80
E_PAD = NW * CPW * CHUNK
ROWS_PT = 640
N_PAD = NS * ROWS_PT
DUMMY = N_PAD - 1
W1AGG = 80
W2AGG = 16
NBURST = 4


def _tc1_body(x_ref, wl_ref, wr_ref, b1_ref, p1_ref, r1_ref):
    x = x_ref[...]
    p1 = jnp.dot(x, wl_ref[...], preferred_element_type=jnp.float32)
    ones = jnp.ones((N, 1), jnp.float32)
    zpad = jnp.zeros((N, W1AGG - H - 1), jnp.float32)
    p1_ref[...] = jnp.concatenate([p1, ones, zpad], axis=1)
    r1_ref[...] = (
        jnp.dot(x, wr_ref[...], preferred_element_type=jnp.float32) + b1_ref[...]
    )


def _tc1(x, W1_l, W1_r, b1):
    return pl.pallas_call(
        _tc1_body,
        out_shape=[
            jax.ShapeDtypeStruct((N, W1AGG), jnp.float32),
            jax.ShapeDtypeStruct((N, H), jnp.float32),
        ],
    )(x, W1_l, W1_r, b1)


def _tc2_body(s1p_ref, r1_ref, w2l_ref, w2r_ref, b2_ref, p2_ref, aux_ref):
    s1 = s1p_ref[0, :N, :] + s1p_ref[1, :N, :]
    deg = jnp.maximum(s1[:, H:H + 1], 1.0)
    rdeg = 1.0 / deg
    h = jnp.maximum(s1[:, :H] * rdeg + r1_ref[...], 0.0)
    p2 = jnp.dot(h, w2l_ref[...], preferred_element_type=jnp.float32)
    r2b = jnp.dot(h, w2r_ref[...], preferred_element_type=jnp.float32) + b2_ref[...]
    p2_ref[...] = jnp.concatenate(
        [p2, jnp.zeros((N, W2AGG - 1), jnp.float32)], axis=1)
    aux_ref[...] = jnp.concatenate(
        [rdeg, r2b, jnp.zeros((N, 6), jnp.float32)], axis=1)


def _tc2(s1p, r1b, W2_l, W2_r, b2):
    return pl.pallas_call(
        _tc2_body,
        out_shape=[
            jax.ShapeDtypeStruct((N, W2AGG), jnp.float32),
            jax.ShapeDtypeStruct((N, 8), jnp.float32),
        ],
    )(s1p, r1b, W2_l, W2_r, b2)


def _tc3_body(s2p_ref, aux_ref, out_ref):
    s2 = s2p_ref[0, :N, 0:1] + s2p_ref[1, :N, 0:1]
    out_ref[...] = s2 * aux_ref[:, 0:1] + aux_ref[:, 1:2]


def _tc3(s2p, aux):
    return pl.pallas_call(
        _tc3_body,
        out_shape=jax.ShapeDtypeStruct((N, 1), jnp.float32),
    )(s2p, aux)


def _make_sc_agg(width):
    mesh = plsc.VectorSubcoreMesh(
        core_axis_name="c", subcore_axis_name="s", num_cores=NC, num_subcores=NS)
    nzero = ROWS_PT // CHUNK

    @functools.partial(
        pl.kernel,
        mesh=mesh,
        compiler_params=pltpu.CompilerParams(use_tc_tiling_on_sc=False),
        out_type=jax.ShapeDtypeStruct((NC, N_PAD, width), jnp.float32),
        scratch_types=[
            pltpu.VMEM((CPW, CHUNK), jnp.int32),
            pltpu.VMEM((CPW, CHUNK), jnp.int32),
            pltpu.VMEM((NBURST, CHUNK, width), jnp.float32),
            pltpu.VMEM_SHARED((N_PAD, width), jnp.float32),
            pltpu.SemaphoreType.DMA,
            pltpu.SemaphoreType.DMA,
        ],
    )
    def sc_agg(table_hbm, src_hbm, dst_hbm, out_hbm, src_v, dst_v, rows,
               acc, gsem, ssem):
        c = lax.axis_index("c")
        s = lax.axis_index("s")
        wid = s * NC + c
        base = s * ROWS_PT

        zbuf = rows.at[0]
        def _zrow(i, _):
            def _zcol(j, _):
                zbuf[i, pl.ds(j * 16, 16)] = jnp.zeros((16,), jnp.float32)
                return 0
            return lax.fori_loop(0, width // 16, _zcol, 0)
        lax.fori_loop(0, CHUNK, _zrow, 0)
        for k in range(nzero):
            pltpu.sync_copy(zbuf, acc.at[pl.ds(base + k * CHUNK, CHUNK), :])
        plsc.subcore_barrier()

        pltpu.sync_copy(src_hbm.at[wid], src_v)
        pltpu.sync_copy(dst_hbm.at[wid], dst_v)

        def _edge_group(g, _):
            jb = g * NBURST
            gd = [pltpu.async_copy(
                table_hbm.at[src_v.at[jb + b]], rows.at[b], gsem)
                for b in range(NBURST)]
            for d in gd:
                d.wait()
            sd = [pltpu.async_copy(
                rows.at[b], acc.at[dst_v.at[jb + b]], ssem, add=True)
                for b in range(NBURST)]
            for d in sd:
                d.wait()
            return 0
        lax.fori_loop(0, CPW // NBURST, _edge_group, 0)
        plsc.subcore_barrier()

        pltpu.sync_copy(acc.at[pl.ds(base, ROWS_PT), :],
                        out_hbm.at[c, pl.ds(base, ROWS_PT), :])

    return sc_agg


_sc_agg_l1 = _make_sc_agg(W1AGG)
_sc_agg_l2 = _make_sc_agg(W2AGG)


def kernel(x, edge_index, W1_l, W1_r, b1, W2_l, W2_r, b2):
    pad = E_PAD - E
    src = jnp.concatenate([edge_index[0], jnp.zeros((pad,), jnp.int32)])
    dst = jnp.concatenate([edge_index[1], jnp.full((pad,), DUMMY, jnp.int32)])
    src = src.reshape(NW, CPW, CHUNK)
    dst = dst.reshape(NW, CPW, CHUNK)

    p1ext, r1b = _tc1(x, W1_l, W1_r, b1.reshape(1, H))
    s1p = _sc_agg_l1(p1ext, src, dst)
    p2ext, aux = _tc2(s1p, r1b, W2_l, W2_r, b2.reshape(1, 1))
    s2p = _sc_agg_l2(p2ext, src, dst)
    return _tc3(s2p, aux)


# speedup vs baseline: 2.3666x; 2.2471x over previous
import functools

import jax
import jax.numpy as jnp
from jax import lax
from jax.experimental import pallas as pl
from jax.experimental.pallas import tpu as pltpu
from jax.experimental.pallas import tpu_sc as plsc

N = 10000
E = 320000
IN = 128
H = 64

NC = 2
NS = 16
NW = NC * NS
CHUNK = 128
CPW = 80
E_PAD = NW * CPW * CHUNK
ROWS_PT = 640
N_PAD = NS * ROWS_PT
DUMMY = N_PAD - 1
W1AGG = 80
W2AGG = 16
NBURST = 2


def _tc1_body(x_ref, wl_ref, wr_ref, b1_ref, p1_ref, r1_ref):
    x = x_ref[...]
    p1 = jnp.dot(x, wl_ref[...], preferred_element_type=jnp.float32)
    p1_ref[...] = jnp.concatenate(
        [p1, jnp.zeros((N_PAD - N, H), jnp.float32)], axis=0)
    r1_ref[...] = (
        jnp.dot(x, wr_ref[...], preferred_element_type=jnp.float32) + b1_ref[...]
    )


def _tc1(x, W1_l, W1_r, b1):
    return pl.pallas_call(
        _tc1_body,
        out_shape=[
            jax.ShapeDtypeStruct((N_PAD, H), jnp.float32),
            jax.ShapeDtypeStruct((N, H), jnp.float32),
        ],
    )(x, W1_l, W1_r, b1)


def _tc2_body(s1p_ref, dp_ref, r1_ref, w2l_ref, w2r_ref, b2_ref, p2_ref,
              aux_ref):
    s1 = s1p_ref[0, :N, :] + s1p_ref[1, :N, :]
    deg = jnp.maximum(dp_ref[0, :N, 0:1] + dp_ref[1, :N, 0:1], 1.0)
    rdeg = 1.0 / deg
    h = jnp.maximum(s1 * rdeg + r1_ref[...], 0.0)
    p2 = jnp.dot(h, w2l_ref[...], preferred_element_type=jnp.float32)
    r2b = jnp.dot(h, w2r_ref[...], preferred_element_type=jnp.float32) + b2_ref[...]
    block = jnp.concatenate(
        [p2, jnp.zeros((N, W2AGG - 1), jnp.float32)], axis=1)
    p2_ref[...] = jnp.concatenate(
        [block, jnp.zeros((N_PAD - N, W2AGG), jnp.float32)], axis=0)
    aux_ref[...] = jnp.concatenate(
        [rdeg, r2b, jnp.zeros((N, 6), jnp.float32)], axis=1)


def _tc2(s1p, dp, r1b, W2_l, W2_r, b2):
    return pl.pallas_call(
        _tc2_body,
        out_shape=[
            jax.ShapeDtypeStruct((N_PAD, W2AGG), jnp.float32),
            jax.ShapeDtypeStruct((N, 8), jnp.float32),
        ],
    )(s1p, dp, r1b, W2_l, W2_r, b2)


def _tc3_body(s2p_ref, aux_ref, out_ref):
    s2 = s2p_ref[0, :N, 0:1] + s2p_ref[1, :N, 0:1]
    out_ref[...] = s2 * aux_ref[:, 0:1] + aux_ref[:, 1:2]


def _tc3(s2p, aux):
    return pl.pallas_call(
        _tc3_body,
        out_shape=jax.ShapeDtypeStruct((N, 1), jnp.float32),
    )(s2p, aux)


DW = 16

_MESH = plsc.VectorSubcoreMesh(
    core_axis_name="c", subcore_axis_name="s", num_cores=NC, num_subcores=NS)
_NZERO = ROWS_PT // CHUNK


def _fill(buf, w, val):
    def _frow(i, _):
        def _fcol(j, _):
            buf[i, pl.ds(j * 16, 16)] = jnp.full((16,), val, jnp.float32)
            return 0
        return lax.fori_loop(0, w // 16, _fcol, 0)
    lax.fori_loop(0, CHUNK, _frow, 0)


@functools.partial(
    pl.kernel,
    mesh=_MESH,
    compiler_params=pltpu.CompilerParams(use_tc_tiling_on_sc=False),
    out_type=jax.ShapeDtypeStruct((NC, N_PAD, DW), jnp.float32),
    scratch_types=[
        pltpu.VMEM((CPW, CHUNK), jnp.int32),
        pltpu.VMEM((CHUNK, DW), jnp.float32),
        pltpu.VMEM_SHARED((N_PAD, DW), jnp.float32),
        pltpu.SemaphoreType.DMA,
    ],
)
def _sc_deg(dst_hbm, dout_hbm, dst_v, obuf, dacc, ssem):
    c = lax.axis_index("c")
    s = lax.axis_index("s")
    wid = s * NC + c
    base = s * ROWS_PT

    _fill(obuf, DW, 0.0)
    for k in range(_NZERO):
        pltpu.sync_copy(obuf, dacc.at[pl.ds(base + k * CHUNK, CHUNK), :])
    _fill(obuf, DW, 1.0)
    pltpu.sync_copy(dst_hbm.at[wid], dst_v)
    plsc.subcore_barrier()

    def _edge_group(g, _):
        jb = g * NBURST
        sd = [pltpu.async_copy(
            obuf, dacc.at[dst_v.at[jb + b]], ssem, add=True)
            for b in range(NBURST)]
        for d in sd:
            d.wait()
        return 0
    lax.fori_loop(0, CPW // NBURST, _edge_group, 0)
    plsc.subcore_barrier()

    pltpu.sync_copy(dacc.at[pl.ds(base, ROWS_PT), :],
                    dout_hbm.at[c, pl.ds(base, ROWS_PT), :])


def _make_sc_agg(width):
    @functools.partial(
        pl.kernel,
        mesh=_MESH,
        compiler_params=pltpu.CompilerParams(use_tc_tiling_on_sc=False),
        out_type=jax.ShapeDtypeStruct((NC, N_PAD, width), jnp.float32),
        scratch_types=[
            pltpu.VMEM((CPW, CHUNK), jnp.int32),
            pltpu.VMEM((CPW, CHUNK), jnp.int32),
            pltpu.VMEM((NBURST, CHUNK, width), jnp.float32),
            pltpu.VMEM_SHARED((N_PAD, width), jnp.float32),
            pltpu.VMEM_SHARED((N_PAD, width), jnp.float32),
            pltpu.SemaphoreType.DMA,
            pltpu.SemaphoreType.DMA,
        ],
    )
    def sc_agg(table_hbm, src_hbm, dst_hbm, out_hbm, src_v, dst_v, rows,
               acc, tbl, gsem, ssem):
        c = lax.axis_index("c")
        s = lax.axis_index("s")
        wid = s * NC + c
        base = s * ROWS_PT

        td = pltpu.async_copy(table_hbm.at[pl.ds(base, ROWS_PT), :],
                              tbl.at[pl.ds(base, ROWS_PT), :], ssem)

        zbuf = rows.at[0]
        _fill(zbuf, width, 0.0)
        for k in range(_NZERO):
            pltpu.sync_copy(zbuf, acc.at[pl.ds(base + k * CHUNK, CHUNK), :])

        pltpu.sync_copy(src_hbm.at[wid], src_v)
        pltpu.sync_copy(dst_hbm.at[wid], dst_v)
        td.wait()
        plsc.subcore_barrier()

        def _edge_group(g, _):
            jb = g * NBURST
            gd = [pltpu.async_copy(
                tbl.at[src_v.at[jb + b]], rows.at[b], gsem)
                for b in range(NBURST)]
            for d in gd:
                d.wait()
            sd = [pltpu.async_copy(
                rows.at[b], acc.at[dst_v.at[jb + b]], ssem, add=True)
                for b in range(NBURST)]
            for d in sd:
                d.wait()
            return 0
        lax.fori_loop(0, CPW // NBURST, _edge_group, 0)
        plsc.subcore_barrier()

        pltpu.sync_copy(acc.at[pl.ds(base, ROWS_PT), :],
                        out_hbm.at[c, pl.ds(base, ROWS_PT), :])

    return sc_agg


_sc_agg_l1 = _make_sc_agg(H)
_sc_agg_l2 = _make_sc_agg(W2AGG)


def kernel(x, edge_index, W1_l, W1_r, b1, W2_l, W2_r, b2):
    pad = E_PAD - E
    src = jnp.concatenate([edge_index[0], jnp.zeros((pad,), jnp.int32)])
    dst = jnp.concatenate([edge_index[1], jnp.full((pad,), DUMMY, jnp.int32)])
    src = src.reshape(NW, CPW, CHUNK)
    dst = dst.reshape(NW, CPW, CHUNK)

    dp = _sc_deg(dst)
    p1, r1b = _tc1(x, W1_l, W1_r, b1.reshape(1, H))
    s1p = _sc_agg_l1(p1, src, dst)
    p2ext, aux = _tc2(s1p, dp, r1b, W2_l, W2_r, b2.reshape(1, 1))
    s2p = _sc_agg_l2(p2ext, src, dst)
    return _tc3(s2p, aux)


# speedup vs baseline: 2.3727x; 1.0026x over previous
import functools

import jax
import jax.numpy as jnp
from jax import lax
from jax.experimental import pallas as pl
from jax.experimental.pallas import tpu as pltpu
from jax.experimental.pallas import tpu_sc as plsc

N = 10000
E = 320000
IN = 128
H = 64

NC = 2
NS = 16
NW = NC * NS
CHUNK = 128
CPW = 80
E_PAD = NW * CPW * CHUNK
EROWS = E // CHUNK
EROWS_PAD = E_PAD // CHUNK
ROWS_PT = 632
N_PAD = NS * ROWS_PT
DUMMY = N_PAD - 1
W2AGG = 16
DW = 16
NBURST = 2


def _tc1_body(x_ref, ei_ref, wl_ref, wr_ref, b1_ref, p1_ref, r1_ref,
              srcp_ref, dstp_ref):
    x = x_ref[...]
    p1 = jnp.dot(x, wl_ref[...], preferred_element_type=jnp.float32)
    p1_ref[...] = jnp.concatenate(
        [p1, jnp.zeros((N_PAD - N, H), jnp.float32)], axis=0)
    r1_ref[...] = (
        jnp.dot(x, wr_ref[...], preferred_element_type=jnp.float32) + b1_ref[...]
    )
    ei = ei_ref[...].reshape(2, EROWS, CHUNK)
    srcp_ref[...] = jnp.concatenate(
        [ei[0], jnp.zeros((EROWS_PAD - EROWS, CHUNK), jnp.int32)], axis=0)
    dstp_ref[...] = jnp.concatenate(
        [ei[1], jnp.full((EROWS_PAD - EROWS, CHUNK), DUMMY, jnp.int32)], axis=0)


def _tc1(x, edge_index, W1_l, W1_r, b1):
    return pl.pallas_call(
        _tc1_body,
        out_shape=[
            jax.ShapeDtypeStruct((N_PAD, H), jnp.float32),
            jax.ShapeDtypeStruct((N, H), jnp.float32),
            jax.ShapeDtypeStruct((EROWS_PAD, CHUNK), jnp.int32),
            jax.ShapeDtypeStruct((EROWS_PAD, CHUNK), jnp.int32),
        ],
    )(x, edge_index, W1_l, W1_r, b1)


def _tc2_body(s1p_ref, dp_ref, r1_ref, w2l_ref, w2r_ref, b2_ref, p2_ref,
              aux_ref):
    s1 = s1p_ref[0, :N, :] + s1p_ref[1, :N, :]
    deg = jnp.maximum(dp_ref[0, :N, 0:1] + dp_ref[1, :N, 0:1], 1.0)
    rdeg = 1.0 / deg
    h = jnp.maximum(s1 * rdeg + r1_ref[...], 0.0)
    p2 = jnp.dot(h, w2l_ref[...], preferred_element_type=jnp.float32)
    r2b = jnp.dot(h, w2r_ref[...], preferred_element_type=jnp.float32) + b2_ref[...]
    block = jnp.concatenate(
        [p2, jnp.zeros((N, W2AGG - 1), jnp.float32)], axis=1)
    p2_ref[...] = jnp.concatenate(
        [block, jnp.zeros((N_PAD - N, W2AGG), jnp.float32)], axis=0)
    aux_ref[...] = jnp.concatenate(
        [rdeg, r2b, jnp.zeros((N, 6), jnp.float32)], axis=1)


def _tc2(s1p, dp, r1b, W2_l, W2_r, b2):
    return pl.pallas_call(
        _tc2_body,
        out_shape=[
            jax.ShapeDtypeStruct((N_PAD, W2AGG), jnp.float32),
            jax.ShapeDtypeStruct((N, 8), jnp.float32),
        ],
    )(s1p, dp, r1b, W2_l, W2_r, b2)


def _tc3_body(s2p_ref, aux_ref, out_ref):
    s2 = s2p_ref[0, :N, 0:1] + s2p_ref[1, :N, 0:1]
    out_ref[...] = s2 * aux_ref[:, 0:1] + aux_ref[:, 1:2]


def _tc3(s2p, aux):
    return pl.pallas_call(
        _tc3_body,
        out_shape=jax.ShapeDtypeStruct((N, 1), jnp.float32),
    )(s2p, aux)


_MESH = plsc.VectorSubcoreMesh(
    core_axis_name="c", subcore_axis_name="s", num_cores=NC, num_subcores=NS)


def _fill(buf, w, val):
    def _frow(i, _):
        def _fcol(j, _):
            buf[i, pl.ds(j * 16, 16)] = jnp.full((16,), val, jnp.float32)
            return 0
        return lax.fori_loop(0, w // 16, _fcol, 0)
    lax.fori_loop(0, CHUNK, _frow, 0)


def _zero_slice(zbuf, dstref, base):
    full, rem = divmod(ROWS_PT, CHUNK)
    for k in range(full):
        pltpu.sync_copy(zbuf, dstref.at[pl.ds(base + k * CHUNK, CHUNK), :])
    if rem:
        pltpu.sync_copy(zbuf.at[pl.ds(0, rem), :],
                        dstref.at[pl.ds(base + full * CHUNK, rem), :])


def _make_sc_agg(width, with_deg):
    out_type = [jax.ShapeDtypeStruct((NC, N_PAD, width), jnp.float32)]
    scratch = [
        pltpu.VMEM((CPW, CHUNK), jnp.int32),
        pltpu.VMEM((CPW, CHUNK), jnp.int32),
        pltpu.VMEM((NBURST, CHUNK, width), jnp.float32),
        pltpu.VMEM_SHARED((N_PAD, width), jnp.float32),
        pltpu.VMEM_SHARED((N_PAD, width), jnp.float32),
        pltpu.SemaphoreType.DMA,
        pltpu.SemaphoreType.DMA,
    ]
    if with_deg:
        out_type.append(jax.ShapeDtypeStruct((NC, N_PAD, DW), jnp.float32))
        scratch += [pltpu.VMEM((CHUNK, DW), jnp.float32),
                    pltpu.VMEM_SHARED((N_PAD, DW), jnp.float32)]

    @functools.partial(
        pl.kernel,
        mesh=_MESH,
        compiler_params=pltpu.CompilerParams(use_tc_tiling_on_sc=False),
        out_type=out_type,
        scratch_types=scratch,
    )
    def sc_agg(table_hbm, src_hbm, dst_hbm, *args):
        if with_deg:
            (out_hbm, dout_hbm, src_v, dst_v, rows, acc, tbl, gsem, ssem,
             obuf, dacc) = args
        else:
            out_hbm, src_v, dst_v, rows, acc, tbl, gsem, ssem = args
        c = lax.axis_index("c")
        s = lax.axis_index("s")
        wid = s * NC + c
        base = s * ROWS_PT

        td = pltpu.async_copy(table_hbm.at[pl.ds(base, ROWS_PT), :],
                              tbl.at[pl.ds(base, ROWS_PT), :], ssem)

        zbuf = rows.at[0]
        _fill(zbuf, width, 0.0)
        _zero_slice(zbuf, acc, base)
        if with_deg:
            _fill(obuf, DW, 0.0)
            _zero_slice(obuf, dacc, base)
            _fill(obuf, DW, 1.0)

        pltpu.sync_copy(src_hbm.at[pl.ds(wid * CPW, CPW), :], src_v)
        pltpu.sync_copy(dst_hbm.at[pl.ds(wid * CPW, CPW), :], dst_v)
        td.wait()
        plsc.subcore_barrier()

        def _edge_group(g, _):
            jb = g * NBURST
            gd = [pltpu.async_copy(
                tbl.at[src_v.at[jb + b]], rows.at[b], gsem)
                for b in range(NBURST)]
            for d in gd:
                d.wait()
            sd = [pltpu.async_copy(
                rows.at[b], acc.at[dst_v.at[jb + b]], ssem, add=True)
                for b in range(NBURST)]
            if with_deg:
                sd += [pltpu.async_copy(
                    obuf, dacc.at[dst_v.at[jb + b]], ssem, add=True)
                    for b in range(NBURST)]
            for d in sd:
                d.wait()
            return 0
        lax.fori_loop(0, CPW // NBURST, _edge_group, 0)
        plsc.subcore_barrier()

        pltpu.sync_copy(acc.at[pl.ds(base, ROWS_PT), :],
                        out_hbm.at[c, pl.ds(base, ROWS_PT), :])
        if with_deg:
            pltpu.sync_copy(dacc.at[pl.ds(base, ROWS_PT), :],
                            dout_hbm.at[c, pl.ds(base, ROWS_PT), :])

    return sc_agg


_sc_agg_l1 = _make_sc_agg(H, True)
_sc_agg_l2 = _make_sc_agg(W2AGG, False)


def kernel(x, edge_index, W1_l, W1_r, b1, W2_l, W2_r, b2):
    p1, r1b, srcp, dstp = _tc1(x, edge_index, W1_l, W1_r, b1.reshape(1, H))
    s1p, dp = _sc_agg_l1(p1, srcp, dstp)
    p2, aux = _tc2(s1p, dp, r1b, W2_l, W2_r, b2.reshape(1, 1))
    (s2p,) = _sc_agg_l2(p2, srcp, dstp)
    return _tc3(s2p, aux)


# speedup vs baseline: 2.7035x; 1.1394x over previous
import functools

import jax
import jax.numpy as jnp
from jax import lax
from jax.experimental import pallas as pl
from jax.experimental.pallas import tpu as pltpu
from jax.experimental.pallas import tpu_sc as plsc

N = 10000
E = 320000
IN = 128
H = 64

NC = 2
NS = 16
NW = NC * NS
CHUNK = 128
CPW = 80
E_PAD = NW * CPW * CHUNK
EROWS = E // CHUNK
EROWS_PAD = E_PAD // CHUNK
ROWS_PT = 632
N_PAD = NS * ROWS_PT
DUMMY = N_PAD - 1
W2AGG = 16
DW = 16
NBURST = 2


def _tc1_body(x_ref, ei_ref, wl_ref, wr_ref, b1_ref, pk1_ref,
              srcp_ref, dstp_ref):
    x = x_ref[...]
    p1 = jnp.dot(x, wl_ref[...], preferred_element_type=jnp.float32)
    r1b = jnp.dot(x, wr_ref[...], preferred_element_type=jnp.float32) + b1_ref[...]
    pk1 = jnp.concatenate([p1, r1b], axis=1)
    pk1_ref[...] = jnp.concatenate(
        [pk1, jnp.zeros((N_PAD - N, 2 * H), jnp.float32)], axis=0)
    ei = ei_ref[...].reshape(2, EROWS, CHUNK)
    srcp_ref[...] = jnp.concatenate(
        [ei[0], jnp.zeros((EROWS_PAD - EROWS, CHUNK), jnp.int32)], axis=0)
    dstp_ref[...] = jnp.concatenate(
        [ei[1], jnp.full((EROWS_PAD - EROWS, CHUNK), DUMMY, jnp.int32)], axis=0)


def _tc1(x, edge_index, W1_l, W1_r, b1):
    return pl.pallas_call(
        _tc1_body,
        out_shape=[
            jax.ShapeDtypeStruct((N_PAD, 2 * H), jnp.float32),
            jax.ShapeDtypeStruct((EROWS_PAD, CHUNK), jnp.int32),
            jax.ShapeDtypeStruct((EROWS_PAD, CHUNK), jnp.int32),
        ],
    )(x, edge_index, W1_l, W1_r, b1)


def _tc2_body(so1_ref, pk1_ref, w2l_ref, w2r_ref, b2_ref, p2_ref, aux_ref):
    s1 = so1_ref[0, :N, :H] + so1_ref[1, :N, :H]
    deg = jnp.maximum(so1_ref[0, :N, H:H + 1] + so1_ref[1, :N, H:H + 1], 1.0)
    rdeg = 1.0 / deg
    h = jnp.maximum(s1 * rdeg + pk1_ref[:N, H:], 0.0)
    p2 = jnp.dot(h, w2l_ref[...], preferred_element_type=jnp.float32)
    r2b = jnp.dot(h, w2r_ref[...], preferred_element_type=jnp.float32) + b2_ref[...]
    block = jnp.concatenate(
        [p2, jnp.zeros((N, 2 * H - 1), jnp.float32)], axis=1)
    p2_ref[...] = jnp.concatenate(
        [block, jnp.zeros((N_PAD - N, 2 * H), jnp.float32)], axis=0)
    aux_ref[...] = jnp.concatenate(
        [rdeg, r2b, jnp.zeros((N, 6), jnp.float32)], axis=1)


def _tc2(so1, pk1, W2_l, W2_r, b2):
    return pl.pallas_call(
        _tc2_body,
        out_shape=[
            jax.ShapeDtypeStruct((N_PAD, 2 * H), jnp.float32),
            jax.ShapeDtypeStruct((N, 8), jnp.float32),
        ],
    )(so1, pk1, W2_l, W2_r, b2)


def _tc3_body(s2p_ref, aux_ref, out_ref):
    s2 = s2p_ref[0, :N, 0:1] + s2p_ref[1, :N, 0:1]
    out_ref[...] = s2 * aux_ref[:, 0:1] + aux_ref[:, 1:2]


def _tc3(s2p, aux):
    return pl.pallas_call(
        _tc3_body,
        out_shape=jax.ShapeDtypeStruct((N, 1), jnp.float32),
    )(s2p, aux)


_MESH = plsc.VectorSubcoreMesh(
    core_axis_name="c", subcore_axis_name="s", num_cores=NC, num_subcores=NS)


def _fill(buf, w, val):
    def _frow(i, _):
        def _fcol(j, _):
            buf[i, pl.ds(j * 16, 16)] = jnp.full((16,), val, jnp.float32)
            return 0
        return lax.fori_loop(0, w // 16, _fcol, 0)
    lax.fori_loop(0, CHUNK, _frow, 0)


def _zero_slice(zbuf, dstref, base):
    full, rem = divmod(ROWS_PT, CHUNK)
    for k in range(full):
        pltpu.sync_copy(zbuf, dstref.at[pl.ds(base + k * CHUNK, CHUNK), :])
    if rem:
        pltpu.sync_copy(zbuf.at[pl.ds(0, rem), :],
                        dstref.at[pl.ds(base + full * CHUNK, rem), :])


def _make_sc_agg(width, with_deg):
    out_type = [jax.ShapeDtypeStruct((NC, N_PAD, 2 * H), jnp.float32)]
    scratch = [
        pltpu.VMEM((CPW, CHUNK), jnp.int32),
        pltpu.VMEM((CPW, CHUNK), jnp.int32),
        pltpu.VMEM((NBURST, CHUNK, width), jnp.float32),
        pltpu.VMEM_SHARED((N_PAD, width), jnp.float32),
        pltpu.VMEM_SHARED((N_PAD, width), jnp.float32),
        pltpu.SemaphoreType.DMA,
        pltpu.SemaphoreType.DMA,
    ]
    if with_deg:
        scratch += [pltpu.VMEM((CHUNK, DW), jnp.float32),
                    pltpu.VMEM_SHARED((N_PAD, DW), jnp.float32)]

    @functools.partial(
        pl.kernel,
        mesh=_MESH,
        compiler_params=pltpu.CompilerParams(use_tc_tiling_on_sc=False),
        out_type=out_type,
        scratch_types=scratch,
    )
    def sc_agg(table_hbm, src_hbm, dst_hbm, *args):
        if with_deg:
            (out_hbm, src_v, dst_v, rows, acc, tbl, gsem, ssem,
             obuf, dacc) = args
        else:
            out_hbm, src_v, dst_v, rows, acc, tbl, gsem, ssem = args
        c = lax.axis_index("c")
        s = lax.axis_index("s")
        wid = s * NC + c
        base = s * ROWS_PT

        td = pltpu.async_copy(
            table_hbm.at[pl.ds(base, ROWS_PT), pl.ds(0, width)],
            tbl.at[pl.ds(base, ROWS_PT), :], ssem)

        zbuf = rows.at[0]
        _fill(zbuf, width, 0.0)
        _zero_slice(zbuf, acc, base)
        if with_deg:
            _fill(obuf, DW, 0.0)
            _zero_slice(obuf, dacc, base)
            _fill(obuf, DW, 1.0)

        pltpu.sync_copy(src_hbm.at[pl.ds(wid * CPW, CPW), :], src_v)
        pltpu.sync_copy(dst_hbm.at[pl.ds(wid * CPW, CPW), :], dst_v)
        td.wait()
        plsc.subcore_barrier()

        def _edge_group(g, _):
            jb = g * NBURST
            gd = [pltpu.async_copy(
                tbl.at[src_v.at[jb + b]], rows.at[b], gsem)
                for b in range(NBURST)]
            for d in gd:
                d.wait()
            sd = [pltpu.async_copy(
                rows.at[b], acc.at[dst_v.at[jb + b]], ssem, add=True)
                for b in range(NBURST)]
            if with_deg:
                sd += [pltpu.async_copy(
                    obuf, dacc.at[dst_v.at[jb + b]], ssem, add=True)
                    for b in range(NBURST)]
            for d in sd:
                d.wait()
            return 0
        lax.fori_loop(0, CPW // NBURST, _edge_group, 0)
        plsc.subcore_barrier()

        pltpu.sync_copy(acc.at[pl.ds(base, ROWS_PT), :],
                        out_hbm.at[c, pl.ds(base, ROWS_PT), pl.ds(0, width)])
        if with_deg:
            pltpu.sync_copy(
                dacc.at[pl.ds(base, ROWS_PT), :],
                out_hbm.at[c, pl.ds(base, ROWS_PT), pl.ds(H, DW)])

    return sc_agg


_sc_agg_l1 = _make_sc_agg(H, True)
_sc_agg_l2 = _make_sc_agg(W2AGG, False)


def kernel(x, edge_index, W1_l, W1_r, b1, W2_l, W2_r, b2):
    pk1, srcp, dstp = _tc1(x, edge_index, W1_l, W1_r, b1.reshape(1, H))
    (so1,) = _sc_agg_l1(pk1, srcp, dstp)
    p2, aux = _tc2(so1, pk1, W2_l, W2_r, b2.reshape(1, 1))
    (s2p,) = _sc_agg_l2(p2, srcp, dstp)
    return _tc3(s2p, aux)
